# baseline (speedup 1.0000x reference)
import jax
import jax.numpy as jnp
from jax.experimental import pallas as pl
from jax.experimental.pallas import tpu as pltpu

_CHUNK = 4000
_NBUF = 8


def _in_copy(x_hbm, in_buf, in_sems, chunk, slot):
    return pltpu.make_async_copy(
        x_hbm.at[pl.ds(chunk * _CHUNK, _CHUNK), :],
        in_buf.at[slot],
        in_sems.at[slot],
    )


def _out_copy(o_hbm, out_buf, out_sems, chunk, slot):
    return pltpu.make_async_copy(
        out_buf.at[slot],
        o_hbm.at[pl.ds(chunk * _CHUNK, _CHUNK), :],
        out_sems.at[slot],
    )


def _body(x_hbm, wt_ref, b_ref, o_hbm, in_buf, out_buf, in_sems, out_sems):
    n_chunks = x_hbm.shape[0] // _CHUNK

    for j in range(_NBUF):
        _in_copy(x_hbm, in_buf, in_sems, j, j).start()

    def step(i, carry):
        slot = jax.lax.rem(i, _NBUF)
        _in_copy(x_hbm, in_buf, in_sems, i, slot).wait()

        @pl.when(i >= _NBUF)
        def _():
            _out_copy(o_hbm, out_buf, out_sems, i - _NBUF, slot).wait()

        out_buf[slot] = (
            jnp.dot(
                in_buf[slot].astype(jnp.bfloat16),
                wt_ref[...],
                preferred_element_type=jnp.float32,
            )
            + b_ref[...]
        )
        _out_copy(o_hbm, out_buf, out_sems, i, slot).start()

        @pl.when(i + _NBUF < n_chunks)
        def _():
            _in_copy(x_hbm, in_buf, in_sems, i + _NBUF, slot).start()

        return carry

    jax.lax.fori_loop(0, n_chunks, step, 0)

    for j in range(_NBUF):
        c = n_chunks - _NBUF + j
        _out_copy(o_hbm, out_buf, out_sems, c, jax.lax.rem(c, _NBUF)).wait()


def kernel(x, W, b):
    n, in_dim = x.shape
    emb_dim = W.shape[0]
    wt = W.T.astype(jnp.bfloat16)
    b2 = b.reshape(1, emb_dim)
    return pl.pallas_call(
        _body,
        in_specs=[
            pl.BlockSpec(memory_space=pl.ANY),
            pl.BlockSpec(memory_space=pltpu.VMEM),
            pl.BlockSpec(memory_space=pltpu.VMEM),
        ],
        out_specs=pl.BlockSpec(memory_space=pl.ANY),
        out_shape=jax.ShapeDtypeStruct((n, emb_dim), jnp.float32),
        scratch_shapes=[
            pltpu.VMEM((_NBUF, _CHUNK, in_dim), jnp.float32),
            pltpu.VMEM((_NBUF, _CHUNK, emb_dim), jnp.float32),
            pltpu.SemaphoreType.DMA((_NBUF,)),
            pltpu.SemaphoreType.DMA((_NBUF,)),
        ],
    )(x, wt, b2)


# speedup vs baseline: 1.0246x; 1.0246x over previous
import jax
import jax.numpy as jnp
from jax.experimental import pallas as pl
from jax.experimental.pallas import tpu as pltpu

_CHUNK = 40000
_NBUF = 2
_PACK = 8


def _in_copy(xp_hbm, in_buf, in_sems, chunk, slot):
    rows = _CHUNK // _PACK
    return pltpu.make_async_copy(
        xp_hbm.at[pl.ds(chunk * rows, rows), :],
        in_buf.at[slot],
        in_sems.at[slot],
    )


def _out_copy(o_hbm, out_buf, out_sems, chunk, slot):
    return pltpu.make_async_copy(
        out_buf.at[slot],
        o_hbm.at[pl.ds(chunk * _CHUNK, _CHUNK), :],
        out_sems.at[slot],
    )


def _body(xp_hbm, wb_ref, b_ref, o_hbm, in_buf, out_buf, in_sems, out_sems):
    n_chunks = o_hbm.shape[0] // _CHUNK

    for j in range(_NBUF):
        _in_copy(xp_hbm, in_buf, in_sems, j, j).start()

    def group(g, carry):
        for j in range(_NBUF):
            i = g * _NBUF + j
            _in_copy(xp_hbm, in_buf, in_sems, i, j).wait()

            @pl.when(i >= _NBUF)
            def _():
                _out_copy(o_hbm, out_buf, out_sems, i - _NBUF, j).wait()

            yp = jnp.dot(
                in_buf[j].astype(jnp.bfloat16),
                wb_ref[...],
                preferred_element_type=jnp.float32,
            )
            out_buf[j] = yp.reshape(_CHUNK, 128) + b_ref[...]
            _out_copy(o_hbm, out_buf, out_sems, i, j).start()

            @pl.when(i + _NBUF < n_chunks)
            def _():
                _in_copy(xp_hbm, in_buf, in_sems, i + _NBUF, j).start()

        return carry

    jax.lax.fori_loop(0, n_chunks // _NBUF, group, 0)

    for j in range(_NBUF):
        _out_copy(o_hbm, out_buf, out_sems, n_chunks - _NBUF + j, j).wait()


def kernel(x, W, b):
    n, in_dim = x.shape
    emb_dim = W.shape[0]
    xp = x.reshape(n // _PACK, _PACK * in_dim)
    wt = W.T.astype(jnp.bfloat16)
    eye8 = jnp.eye(_PACK, dtype=jnp.bfloat16)
    wb = (eye8[:, None, :, None] * wt[None, :, None, :]).reshape(
        _PACK * in_dim, _PACK * emb_dim
    )
    b2 = b.reshape(1, emb_dim)
    return pl.pallas_call(
        _body,
        in_specs=[
            pl.BlockSpec(memory_space=pl.ANY),
            pl.BlockSpec(memory_space=pltpu.VMEM),
            pl.BlockSpec(memory_space=pltpu.VMEM),
        ],
        out_specs=pl.BlockSpec(memory_space=pl.ANY),
        out_shape=jax.ShapeDtypeStruct((n, emb_dim), jnp.float32),
        scratch_shapes=[
            pltpu.VMEM((_NBUF, _CHUNK // _PACK, _PACK * in_dim), jnp.float32),
            pltpu.VMEM((_NBUF, _CHUNK, emb_dim), jnp.float32),
            pltpu.SemaphoreType.DMA((_NBUF,)),
            pltpu.SemaphoreType.DMA((_NBUF,)),
        ],
    )(xp, wb, b2)
